# baseline (speedup 1.0000x reference)
import functools

import jax
import jax.numpy as jnp
from jax import lax
from jax.experimental import pallas as pl
from jax.experimental.pallas import tpu as pltpu
from jax.experimental.pallas import tpu_sc as plsc

B = 16
K = 128
HW = 128 * 128
NTILES = 32
PAIRS = (B * K) // NTILES
CHANS = (2, 1, 3, 8)
NCH = sum(CHANS)
_CH_MAP = [(a, c) for a, n in enumerate(CHANS) for c in range(n)]
_STREAMS = [(0, 2), (2, 1), (3, 2), (5, 1), (6, 2), (8, 2), (10, 2), (12, 2)]


def _dep_transform(x):
    s = 1.0 / (1.0 + jnp.exp(-x))
    return 1.0 / (s + 1e-6) - 1.0


def _sc_gather(ind_flat, regf, depf, dimf, rotf):
    n128 = sum(1 for _, n in _STREAMS if n == 2)
    n64 = len(_STREAMS) - n128
    mesh = plsc.VectorSubcoreMesh(core_axis_name="c", subcore_axis_name="s")

    @functools.partial(
        pl.kernel,
        mesh=mesh,
        out_type=jax.ShapeDtypeStruct((NTILES * NCH * PAIRS,), jnp.float32),
        scratch_types=[
            pltpu.VMEM((PAIRS,), jnp.int32),
            pltpu.VMEM((n128, 128), jnp.int32),
            pltpu.VMEM((n64, PAIRS), jnp.int32),
            pltpu.VMEM((NCH * PAIRS,), jnp.float32),
            pltpu.SemaphoreType.DMA,
        ],
    )
    def sc_kernel(ind_hbm, reg_hbm, dep_hbm, dim_hbm, rot_hbm,
                  out_hbm, ind_v, idxw_v, idxn_v, outb_v, sem):
        wid = lax.axis_index("c") * 16 + lax.axis_index("s")
        b = wid // 2
        base = wid * PAIRS

        pltpu.sync_copy(ind_hbm.at[pl.ds(base, PAIRS)], ind_v)

        srcs = (reg_hbm, dep_hbm, dim_hbm, rot_hbm)
        iw = inr = 0
        stream_refs = []
        for ch0, n_ch in _STREAMS:
            if n_ch == 2:
                row, iw = iw, iw + 1
                idx_ref = idxw_v.at[row]
            else:
                row, inr = inr, inr + 1
                idx_ref = idxn_v.at[row]
            for t in range(n_ch):
                ai, c = _CH_MAP[ch0 + t]
                off = (b * CHANS[ai] + c) * HW
                for j in range(PAIRS // 16):
                    sl = pl.ds(64 * t + 16 * j, 16)
                    if n_ch == 2:
                        idxw_v[row, sl] = ind_v[pl.ds(16 * j, 16)] + off
                    else:
                        idxn_v[row, sl] = ind_v[pl.ds(16 * j, 16)] + off
            stream_refs.append((srcs[_CH_MAP[ch0][0]], idx_ref,
                                ch0 * PAIRS, n_ch * PAIRS))

        handles = [
            pltpu.async_copy(src.at[idx_ref], outb_v.at[pl.ds(o, ln)], sem)
            for src, idx_ref, o, ln in stream_refs]
        for h in handles:
            h.wait()

        pltpu.sync_copy(outb_v, out_hbm.at[pl.ds(wid * NCH * PAIRS,
                                                 NCH * PAIRS)])

    return sc_kernel(ind_flat, regf, depf, dimf, rotf)


_NB = 4
_BR = (B * 3 * 128) // _NB


def _mse_body(hm_ref, t_ref, tm_ref, o_ref):
    del tm_ref
    i = pl.program_id(0)
    x = hm_ref[...]
    t = t_ref[...]
    s = jnp.clip(1.0 / (1.0 + jnp.exp(-x)), 1e-4, 1.0 - 1e-4)
    d = s - t
    ps = jnp.sum(d * d, axis=0, keepdims=True)

    @pl.when(i == 0)
    def _init():
        o_ref[...] = ps

    @pl.when(i > 0)
    def _acc():
        o_ref[...] += ps


def _tc_mse(hm2, hmt2, tmask):
    return pl.pallas_call(
        _mse_body,
        grid=(_NB,),
        in_specs=[
            pl.BlockSpec((_BR, 128), lambda i: (i, 0)),
            pl.BlockSpec((_BR, 128), lambda i: (i, 0)),
            pl.BlockSpec((NTILES, NCH * PAIRS), lambda i: (0, 0)),
        ],
        out_specs=pl.BlockSpec((1, 128), lambda i: (0, 0)),
        out_shape=jax.ShapeDtypeStruct((1, 128), jnp.float32),
    )(hm2, hmt2, tmask)


_CH0GL = [0, 2, 3, 6, NCH]


def _combine_body(mse_ref, pr_ref, tg_ref, mf_ref, o_tot, o_hm, o_off, o_dep,
                  o_dim, o_rot):
    pr = pr_ref[...]
    tg = tg_ref[...]
    mf = mf_ref[...]
    row = lax.broadcasted_iota(jnp.int32, pr.shape, 0)
    col = lax.broadcasted_iota(jnp.int32, pr.shape, 1)
    ch = 2 * (row % 7) + (col >= 64).astype(jnp.int32)
    pd = jnp.where(ch == 2, _dep_transform(pr), pr)
    d = jnp.abs(pd * mf - tg)
    hm_l = jnp.sum(mse_ref[...]) / (B * 3.0 * HW)
    ls = []
    for a in range(4):
        sel = jnp.logical_and(ch >= _CH0GL[a], ch < _CH0GL[a + 1])
        ls.append(jnp.sum(jnp.where(sel, d, 0.0)) / (B * K * float(CHANS[a])))
    o_hm[0, 0] = hm_l
    o_off[0, 0] = ls[0]
    o_dep[0, 0] = ls[1]
    o_dim[0, 0] = ls[2]
    o_rot[0, 0] = ls[3]
    o_tot[0, 0] = hm_l + ls[0] + ls[1] + ls[2] + ls[3]


def _tc_combine(mse, preds2d, tmask, maskf):
    scalar = jax.ShapeDtypeStruct((1, 1), jnp.float32)
    return pl.pallas_call(
        _combine_body,
        out_specs=[pl.BlockSpec(memory_space=pltpu.SMEM)] * 6,
        out_shape=[scalar] * 6,
    )(mse, preds2d, tmask, maskf)


def _to_tile_layout(t):
    C = t.shape[2]
    return t.transpose(0, 2, 1).reshape(B, C, 2, PAIRS).transpose(
        0, 2, 1, 3).reshape(NTILES, C * PAIRS)


def kernel(hm, reg, dep, dim, rot, hm_target, reg_mask, ind, reg_target,
           dep_target, dim_target, rot_target):
    ind_flat = ind.astype(jnp.int32).reshape(-1)
    preds = _sc_gather(ind_flat, reg.reshape(-1), dep.reshape(-1),
                       dim.reshape(-1), rot.reshape(-1))
    mf3 = reg_mask.astype(jnp.float32)[:, :, None]
    tmask = jnp.concatenate(
        [_to_tile_layout(t * mf3) for t in
         (reg_target, dep_target, dim_target, rot_target)], axis=1)
    mse = _tc_mse(hm.reshape(B * 3 * 128, 128),
                  hm_target.reshape(B * 3 * 128, 128), tmask)
    maskf = jnp.broadcast_to(
        reg_mask.astype(jnp.float32).reshape(B, 2, 1, 1, PAIRS),
        (B, 2, 7, 2, PAIRS)).reshape(NTILES * 7, 128)
    outs = _tc_combine(mse, preds.reshape(NTILES * 7, 128),
                       tmask.reshape(NTILES * 7, 128), maskf)
    tot, hm_l, off_l, dep_l, dim_l, rot_l = [o.reshape(()) for o in outs]
    return (tot, hm_l, off_l, dep_l, dim_l, rot_l)


# speedup vs baseline: 1.0301x; 1.0301x over previous
import functools

import jax
import jax.numpy as jnp
from jax import lax
from jax.experimental import pallas as pl
from jax.experimental.pallas import tpu as pltpu
from jax.experimental.pallas import tpu_sc as plsc

B = 16
K = 128
HW = 128 * 128
NTILES = 32
PAIRS = (B * K) // NTILES
CHANS = (2, 1, 3, 8)
NCH = sum(CHANS)
_CH_MAP = [(a, c) for a, n in enumerate(CHANS) for c in range(n)]
_STREAMS = [(0, 2), (2, 1), (3, 2), (5, 1), (6, 2), (8, 2), (10, 2), (12, 2)]


def _dep_transform(x):
    s = 1.0 / (1.0 + jnp.exp(-x))
    return 1.0 / (s + 1e-6) - 1.0


def _sc_gather(ind_flat, mask_flat, regf, depf, dimf, rotf):
    n128 = sum(1 for _, n in _STREAMS if n == 2)
    n64 = len(_STREAMS) - n128
    mesh = plsc.VectorSubcoreMesh(core_axis_name="c", subcore_axis_name="s")

    @functools.partial(
        pl.kernel,
        mesh=mesh,
        out_type=jax.ShapeDtypeStruct((NTILES * NCH * PAIRS,), jnp.float32),
        scratch_types=[
            pltpu.VMEM((PAIRS,), jnp.int32),
            pltpu.VMEM((PAIRS,), jnp.int32),
            pltpu.VMEM((n128, 128), jnp.int32),
            pltpu.VMEM((n64, PAIRS), jnp.int32),
            pltpu.VMEM((NCH * PAIRS,), jnp.float32),
            pltpu.SemaphoreType.DMA,
        ],
    )
    def sc_kernel(ind_hbm, mask_hbm, reg_hbm, dep_hbm, dim_hbm, rot_hbm,
                  out_hbm, ind_v, mask_v, idxw_v, idxn_v, outb_v, sem):
        wid = lax.axis_index("c") * 16 + lax.axis_index("s")
        b = wid // 2
        base = wid * PAIRS

        pltpu.sync_copy(ind_hbm.at[pl.ds(base, PAIRS)], ind_v)
        pltpu.sync_copy(mask_hbm.at[pl.ds(base, PAIRS)], mask_v)

        srcs = (reg_hbm, dep_hbm, dim_hbm, rot_hbm)
        iw = inr = 0
        stream_refs = []
        for ch0, n_ch in _STREAMS:
            if n_ch == 2:
                row, iw = iw, iw + 1
                idx_ref = idxw_v.at[row]
            else:
                row, inr = inr, inr + 1
                idx_ref = idxn_v.at[row]
            for t in range(n_ch):
                ai, c = _CH_MAP[ch0 + t]
                off = (b * CHANS[ai] + c) * HW
                for j in range(PAIRS // 16):
                    sl = pl.ds(64 * t + 16 * j, 16)
                    if n_ch == 2:
                        idxw_v[row, sl] = ind_v[pl.ds(16 * j, 16)] + off
                    else:
                        idxn_v[row, sl] = ind_v[pl.ds(16 * j, 16)] + off
            stream_refs.append((srcs[_CH_MAP[ch0][0]], idx_ref,
                                ch0 * PAIRS, n_ch * PAIRS))

        handles = [
            pltpu.async_copy(src.at[idx_ref], outb_v.at[pl.ds(o, ln)], sem)
            for src, idx_ref, o, ln in stream_refs]
        for h in handles:
            h.wait()

        for j in range(PAIRS // 16):
            sl = pl.ds(2 * PAIRS + 16 * j, 16)
            outb_v[sl] = _dep_transform(outb_v[sl])
        for j in range(PAIRS // 16):
            mf = mask_v[pl.ds(16 * j, 16)].astype(jnp.float32)
            for ch in range(NCH):
                sl = pl.ds(ch * PAIRS + 16 * j, 16)
                outb_v[sl] = outb_v[sl] * mf

        pltpu.sync_copy(outb_v, out_hbm.at[pl.ds(wid * NCH * PAIRS,
                                                 NCH * PAIRS)])

    return sc_kernel(ind_flat, mask_flat, regf, depf, dimf, rotf)


_NB = 4
_BR = (B * 3 * 128) // _NB


def _mse_body(hm_ref, t_ref, tm_ref, o_ref):
    del tm_ref
    i = pl.program_id(0)
    x = hm_ref[...]
    t = t_ref[...]
    s = jnp.clip(1.0 / (1.0 + jnp.exp(-x)), 1e-4, 1.0 - 1e-4)
    d = s - t
    ps = jnp.sum(d * d, axis=0, keepdims=True)

    @pl.when(i == 0)
    def _init():
        o_ref[...] = ps

    @pl.when(i > 0)
    def _acc():
        o_ref[...] += ps


def _tc_mse(hm2, hmt2, tmask):
    return pl.pallas_call(
        _mse_body,
        grid=(_NB,),
        in_specs=[
            pl.BlockSpec((_BR, 128), lambda i: (i, 0)),
            pl.BlockSpec((_BR, 128), lambda i: (i, 0)),
            pl.BlockSpec((NTILES, NCH * PAIRS), lambda i: (0, 0)),
        ],
        out_specs=pl.BlockSpec((1, 128), lambda i: (0, 0)),
        out_shape=jax.ShapeDtypeStruct((1, 128), jnp.float32),
    )(hm2, hmt2, tmask)


_CH0GL = [0, 2, 3, 6, NCH]


def _combine_body(mse_ref, pr_ref, tg_ref, o_tot, o_hm, o_off, o_dep,
                  o_dim, o_rot):
    pr = pr_ref[...]
    tg = tg_ref[...]
    row = lax.broadcasted_iota(jnp.int32, pr.shape, 0)
    col = lax.broadcasted_iota(jnp.int32, pr.shape, 1)
    ch = 2 * (row % 7) + (col >= 64).astype(jnp.int32)
    d = jnp.abs(pr - tg)
    hm_l = jnp.sum(mse_ref[...]) / (B * 3.0 * HW)
    ls = []
    for a in range(4):
        sel = jnp.logical_and(ch >= _CH0GL[a], ch < _CH0GL[a + 1])
        ls.append(jnp.sum(jnp.where(sel, d, 0.0)) / (B * K * float(CHANS[a])))
    o_hm[0, 0] = hm_l
    o_off[0, 0] = ls[0]
    o_dep[0, 0] = ls[1]
    o_dim[0, 0] = ls[2]
    o_rot[0, 0] = ls[3]
    o_tot[0, 0] = hm_l + ls[0] + ls[1] + ls[2] + ls[3]


def _tc_combine(mse, preds2d, tmask):
    scalar = jax.ShapeDtypeStruct((1, 1), jnp.float32)
    return pl.pallas_call(
        _combine_body,
        out_specs=[pl.BlockSpec(memory_space=pltpu.SMEM)] * 6,
        out_shape=[scalar] * 6,
    )(mse, preds2d, tmask)


def _to_tile_layout(t):
    C = t.shape[2]
    return t.transpose(0, 2, 1).reshape(B, C, 2, PAIRS).transpose(
        0, 2, 1, 3).reshape(NTILES, C * PAIRS)


def kernel(hm, reg, dep, dim, rot, hm_target, reg_mask, ind, reg_target,
           dep_target, dim_target, rot_target):
    ind_flat = ind.astype(jnp.int32).reshape(-1)
    mask_flat = reg_mask.astype(jnp.int32).reshape(-1)
    preds = _sc_gather(ind_flat, mask_flat, reg.reshape(-1), dep.reshape(-1),
                       dim.reshape(-1), rot.reshape(-1))
    mf3 = reg_mask.astype(jnp.float32)[:, :, None]
    tmask = jnp.concatenate(
        [_to_tile_layout(t * mf3) for t in
         (reg_target, dep_target, dim_target, rot_target)], axis=1)
    mse = _tc_mse(hm.reshape(B * 3 * 128, 128),
                  hm_target.reshape(B * 3 * 128, 128), tmask)
    outs = _tc_combine(mse, preds.reshape(NTILES * 7, 128),
                       tmask.reshape(NTILES * 7, 128))
    tot, hm_l, off_l, dep_l, dim_l, rot_l = [o.reshape(()) for o in outs]
    return (tot, hm_l, off_l, dep_l, dim_l, rot_l)


# speedup vs baseline: 1.0414x; 1.0110x over previous
import functools

import jax
import jax.numpy as jnp
from jax import lax
from jax.experimental import pallas as pl
from jax.experimental.pallas import tpu as pltpu
from jax.experimental.pallas import tpu_sc as plsc

B = 16
K = 128
HW = 128 * 128
NTILES = 32
PAIRS = (B * K) // NTILES
CHANS = (2, 1, 3, 8)
NCH = sum(CHANS)
_CH_MAP = [(a, c) for a, n in enumerate(CHANS) for c in range(n)]
_STREAMS = [(0, 2), (2, 1), (3, 2), (5, 1), (6, 2), (8, 2), (10, 2), (12, 2)]


def _dep_transform(x):
    s = 1.0 / (1.0 + jnp.exp(-x))
    return 1.0 / (s + 1e-6) - 1.0


def _sc_gather(ind_flat, mask_flat, regf, depf, dimf, rotf):
    n128 = sum(1 for _, n in _STREAMS if n == 2)
    n64 = len(_STREAMS) - n128
    mesh = plsc.VectorSubcoreMesh(core_axis_name="c", subcore_axis_name="s")

    @functools.partial(
        pl.kernel,
        mesh=mesh,
        out_type=jax.ShapeDtypeStruct((NTILES * NCH * PAIRS,), jnp.float32),
        scratch_types=[
            pltpu.VMEM((PAIRS,), jnp.int32),
            pltpu.VMEM((PAIRS,), jnp.int32),
            pltpu.VMEM((n128, 128), jnp.int32),
            pltpu.VMEM((n64, PAIRS), jnp.int32),
            pltpu.VMEM((NCH * PAIRS,), jnp.float32),
            pltpu.SemaphoreType.DMA,
        ],
    )
    def sc_kernel(ind_hbm, mask_hbm, reg_hbm, dep_hbm, dim_hbm, rot_hbm,
                  out_hbm, ind_v, mask_v, idxw_v, idxn_v, outb_v, sem):
        wid = lax.axis_index("c") * 16 + lax.axis_index("s")
        b = wid // 2
        base = wid * PAIRS

        pltpu.sync_copy(ind_hbm.at[pl.ds(base, PAIRS)], ind_v)
        pltpu.sync_copy(mask_hbm.at[pl.ds(base, PAIRS)], mask_v)

        srcs = (reg_hbm, dep_hbm, dim_hbm, rot_hbm)
        iw = inr = 0
        stream_refs = []
        for ch0, n_ch in _STREAMS:
            if n_ch == 2:
                row, iw = iw, iw + 1
                idx_ref = idxw_v.at[row]
            else:
                row, inr = inr, inr + 1
                idx_ref = idxn_v.at[row]
            for t in range(n_ch):
                ai, c = _CH_MAP[ch0 + t]
                off = (b * CHANS[ai] + c) * HW
                for j in range(PAIRS // 16):
                    sl = pl.ds(64 * t + 16 * j, 16)
                    if n_ch == 2:
                        idxw_v[row, sl] = ind_v[pl.ds(16 * j, 16)] + off
                    else:
                        idxn_v[row, sl] = ind_v[pl.ds(16 * j, 16)] + off
            stream_refs.append((srcs[_CH_MAP[ch0][0]], idx_ref,
                                ch0 * PAIRS, n_ch * PAIRS))

        handles = [
            pltpu.async_copy(src.at[idx_ref], outb_v.at[pl.ds(o, ln)], sem)
            for src, idx_ref, o, ln in stream_refs]
        for h in handles:
            h.wait()

        for j in range(PAIRS // 16):
            sl = pl.ds(2 * PAIRS + 16 * j, 16)
            outb_v[sl] = _dep_transform(outb_v[sl])
        for j in range(PAIRS // 16):
            mf = mask_v[pl.ds(16 * j, 16)].astype(jnp.float32)
            for ch in range(NCH):
                sl = pl.ds(ch * PAIRS + 16 * j, 16)
                outb_v[sl] = outb_v[sl] * mf

        pltpu.sync_copy(outb_v, out_hbm.at[pl.ds(wid * NCH * PAIRS,
                                                 NCH * PAIRS)])

    return sc_kernel(ind_flat, mask_flat, regf, depf, dimf, rotf)


_NB = 4
_BR = (B * 3 * 128) // _NB


def _mse_body(hm_ref, t_ref, o_ref):
    i = pl.program_id(0)
    x = hm_ref[...]
    t = t_ref[...]
    s = jnp.clip(1.0 / (1.0 + jnp.exp(-x)), 1e-4, 1.0 - 1e-4)
    d = s - t
    ps = jnp.sum(d * d, axis=0, keepdims=True)

    @pl.when(i == 0)
    def _init():
        o_ref[...] = ps

    @pl.when(i > 0)
    def _acc():
        o_ref[...] += ps


def _tc_mse(hm2, hmt2):
    return pl.pallas_call(
        _mse_body,
        grid=(_NB,),
        in_specs=[
            pl.BlockSpec((_BR, 128), lambda i: (i, 0)),
            pl.BlockSpec((_BR, 128), lambda i: (i, 0)),
        ],
        out_specs=pl.BlockSpec((1, 128), lambda i: (0, 0)),
        out_shape=jax.ShapeDtypeStruct((1, 128), jnp.float32),
    )(hm2, hmt2)


_CH0GL = [0, 2, 3, 6, NCH]


def _combine_body(mse_ref, pr_ref, tg_ref, o_tot, o_hm, o_off, o_dep,
                  o_dim, o_rot):
    pr = pr_ref[...]
    tg = tg_ref[...]
    row = lax.broadcasted_iota(jnp.int32, pr.shape, 0)
    col = lax.broadcasted_iota(jnp.int32, pr.shape, 1)
    ch = 2 * (row % 7) + (col >= 64).astype(jnp.int32)
    d = jnp.abs(pr - tg)
    hm_l = jnp.sum(mse_ref[...]) / (B * 3.0 * HW)
    ls = []
    for a in range(4):
        sel = jnp.logical_and(ch >= _CH0GL[a], ch < _CH0GL[a + 1])
        ls.append(jnp.sum(jnp.where(sel, d, 0.0)) / (B * K * float(CHANS[a])))
    o_hm[0, 0] = hm_l
    o_off[0, 0] = ls[0]
    o_dep[0, 0] = ls[1]
    o_dim[0, 0] = ls[2]
    o_rot[0, 0] = ls[3]
    o_tot[0, 0] = hm_l + ls[0] + ls[1] + ls[2] + ls[3]


def _tc_combine(mse, preds2d, tmask):
    scalar = jax.ShapeDtypeStruct((1, 1), jnp.float32)
    return pl.pallas_call(
        _combine_body,
        out_specs=[pl.BlockSpec(memory_space=pltpu.SMEM)] * 6,
        out_shape=[scalar] * 6,
    )(mse, preds2d, tmask)


def _to_tile_layout(t):
    C = t.shape[2]
    return t.transpose(0, 2, 1).reshape(B, C, 2, PAIRS).transpose(
        0, 2, 1, 3).reshape(NTILES, C * PAIRS)


def kernel(hm, reg, dep, dim, rot, hm_target, reg_mask, ind, reg_target,
           dep_target, dim_target, rot_target):
    ind_flat = ind.astype(jnp.int32).reshape(-1)
    mask_flat = reg_mask.astype(jnp.int32).reshape(-1)
    preds = _sc_gather(ind_flat, mask_flat, reg.reshape(-1), dep.reshape(-1),
                       dim.reshape(-1), rot.reshape(-1))
    mf3 = reg_mask.astype(jnp.float32)[:, :, None]
    tmask = jnp.concatenate(
        [_to_tile_layout(t * mf3) for t in
         (reg_target, dep_target, dim_target, rot_target)], axis=1)
    mse = _tc_mse(hm.reshape(B * 3 * 128, 128),
                  hm_target.reshape(B * 3 * 128, 128))
    outs = _tc_combine(mse, preds.reshape(NTILES * 7, 128),
                       tmask.reshape(NTILES * 7, 128))
    tot, hm_l, off_l, dep_l, dim_l, rot_l = [o.reshape(()) for o in outs]
    return (tot, hm_l, off_l, dep_l, dim_l, rot_l)
